# baseline (speedup 1.0000x reference)
import jax
import jax.numpy as jnp
from jax.experimental import pallas as pl
from jax.experimental.pallas import tpu as pltpu

_K = 256
_N = 4096
_BIG = 1e30
_C = 8
_NB = _N // _C
_LANES = 128
_STEPS = _NB + 2 * (_LANES - 1) + 2


def _dtw_body(xp_ref, kr_ref, out_ref):
    k0 = kr_ref[0:1, :]
    k1 = kr_ref[1:2, :]
    lane = jax.lax.broadcasted_iota(jnp.int32, (1, _LANES), 1)
    lane0 = lane == 0

    def body(s, carry):
        WA, WB, btS, btR, d0L, d1L, corner = carry

        mfirst = (2 * lane) == s
        d0L = jnp.where(mfirst, _BIG, d0L)
        d1L = jnp.where(mfirst, _BIG, d1L)
        corner = jnp.where(mfirst, _BIG, corner)
        corner = jnp.where(jnp.logical_and(lane0, s == 0), 0.0, corner)

        btS_n = []
        for t in range(_C):
            rolled = pltpu.roll(btR[t], 1, 1)

            @pl.when(s >= 2 * _LANES - 1)
            def _(rolled=rolled, t=t):
                out_ref[pl.ds((s - (2 * _LANES - 1)) * _C + t, 1), :] = rolled

            btS_n.append(jnp.where(lane0, 0.0, rolled))
        btS_n = tuple(btS_n)

        WA_n = []
        for t in range(_C):
            xi = xp_ref[jnp.minimum((s + 2) * _C + t, _N - 1)]
            WA_n.append(jnp.where(lane0, xi, pltpu.roll(WA[t], 1, 1)))
        WA_n = tuple(WA_n)

        u_prev = corner
        d0 = d0L
        d1 = d1L
        btR_n = []
        for t in range(_C):
            up = btS[t]
            c0 = k0 - WA[t]
            c0 = c0 * c0
            c1 = k1 - WA[t]
            c1 = c1 * c1
            nd0 = c0 + jnp.minimum(jnp.minimum(up, u_prev), d0)
            nd1 = c1 + jnp.minimum(jnp.minimum(nd0, d0), d1)
            btR_n.append(nd1)
            u_prev = up
            d0 = nd0
            d1 = nd1
        btR_n = tuple(btR_n)

        return (WB, WA_n, btS_n, btR_n, d0, d1, u_prev)

    lz = k0 * 0.0
    big = lz + _BIG
    zl = jnp.where(lane0, lz, big)
    WA0 = tuple(lz + xp_ref[t] for t in range(_C))
    WB0 = tuple(lz + xp_ref[_C + t] for t in range(_C))
    btS0 = tuple(zl for _ in range(_C))
    btR0 = tuple(big for _ in range(_C))
    jax.lax.fori_loop(0, _STEPS, body, (WA0, WB0, btS0, btR0, big, big, zl))


def _run(x, kern, interpret=False):
    kr = kern.reshape(_LANES, 2).T
    out = pl.pallas_call(
        _dtw_body,
        in_specs=[
            pl.BlockSpec(memory_space=pltpu.SMEM),
            pl.BlockSpec(memory_space=pltpu.VMEM),
        ],
        out_shape=jax.ShapeDtypeStruct((_N, _LANES), jnp.float32),
        interpret=interpret,
    )(x, kr)
    return out[:, 0]


def kernel(x, kernel):
    return _run(x, kernel)


# speedup vs baseline: 6.8197x; 6.8197x over previous
import jax
import jax.numpy as jnp
from jax.experimental import pallas as pl
from jax.experimental.pallas import tpu as pltpu

_K = 256
_N = 4096
_BIG = 1e30
_C = 8
_NB = _N // _C
_LANES = 128
_STEPS = _NB + 2 * (_LANES - 1) + 1


def _dtw_body(xp_ref, kr_ref, out_ref):
    k0 = kr_ref[0:1, :]
    k1 = kr_ref[1:2, :]
    lane = jax.lax.broadcasted_iota(jnp.int32, (1, _LANES), 1)
    lane0 = lane == 0

    def body(s, carry):
        WA, WB, btS, btR, d0L, d1L, corner = carry

        mfirst = (2 * lane) == s
        d0L = jnp.where(mfirst, _BIG, d0L)
        d1L = jnp.where(mfirst, _BIG, d1L)
        corner = jnp.where(mfirst, _BIG, corner)
        corner = jnp.where(jnp.logical_and(lane0, s == 0), 0.0, corner)

        base = (s - (2 * _LANES - 1)) * _C
        btS_n = []
        for t in range(_C):
            rolled = pltpu.roll(btR[t], 1, 1)
            out_ref[pl.ds(jnp.maximum(base + t, 0), 1), :] = rolled
            btS_n.append(jnp.where(lane0, 0.0, rolled))
        btS_n = tuple(btS_n)

        WA_n = []
        for t in range(_C):
            xi = xp_ref[jnp.minimum((s + 2) * _C + t, _N - 1)]
            WA_n.append(jnp.where(lane0, xi, pltpu.roll(WA[t], 1, 1)))
        WA_n = tuple(WA_n)

        u_prev = corner
        d0 = d0L
        d1 = d1L
        btR_n = []
        for t in range(_C):
            up = btS[t]
            c0 = k0 - WA[t]
            c0 = c0 * c0
            c1 = k1 - WA[t]
            c1 = c1 * c1
            nd0 = c0 + jnp.minimum(jnp.minimum(up, u_prev), d0)
            nd1 = c1 + jnp.minimum(jnp.minimum(nd0, d0), d1)
            btR_n.append(nd1)
            u_prev = up
            d0 = nd0
            d1 = nd1
        btR_n = tuple(btR_n)

        return (WB, WA_n, btS_n, btR_n, d0, d1, u_prev)

    lz = k0 * 0.0
    big = lz + _BIG
    zl = jnp.where(lane0, lz, big)
    WA0 = tuple(lz + xp_ref[t] for t in range(_C))
    WB0 = tuple(lz + xp_ref[_C + t] for t in range(_C))
    btS0 = tuple(zl for _ in range(_C))
    btR0 = tuple(big for _ in range(_C))
    jax.lax.fori_loop(0, _STEPS, body, (WA0, WB0, btS0, btR0, big, big, zl))


def _run(x, kern, interpret=False):
    kr = kern.reshape(_LANES, 2).T
    out = pl.pallas_call(
        _dtw_body,
        in_specs=[
            pl.BlockSpec(memory_space=pltpu.SMEM),
            pl.BlockSpec(memory_space=pltpu.VMEM),
        ],
        out_shape=jax.ShapeDtypeStruct((_N, _LANES), jnp.float32),
        interpret=interpret,
    )(x, kr)
    return out[:, 0]


def kernel(x, kernel):
    return _run(x, kernel)
